# Initial kernel scaffold; baseline (speedup 1.0000x reference)
import functools

import jax
import jax.numpy as jnp
from jax import lax
from jax.experimental import pallas as pl
from jax.experimental.pallas import tpu as pltpu
from jax.experimental.pallas import tpu_sc as plsc

N = 10000
E = 160000
D = 256

NC = 2
NS = 16
NH = N // NC
NH_PAD = 5120
DUMMY = NH_PAD
AGG_ROWS = NH_PAD + 8
EPT = E // NS
K = 80
NCHUNK = EPT // K

_mesh = plsc.VectorSubcoreMesh(core_axis_name="c", subcore_axis_name="s")


def _zero_rows(rows):
    z = jnp.zeros((16,), jnp.float32)
    r_dim, c_dim = rows.shape

    @pl.loop(0, r_dim)
    def _(r):
        for k in range(c_dim // 16):
            rows[r, pl.ds(16 * k, 16)] = z


@functools.partial(
    pl.kernel,
    out_type=(
        jax.ShapeDtypeStruct((NS * 640,), jnp.float32),
        jax.ShapeDtypeStruct((NC, NH_PAD), jnp.float32),
    ),
    mesh=_mesh,
    scratch_types=[
        pltpu.VMEM((EPT,), jnp.int32),
        pltpu.VMEM((EPT,), jnp.int32),
        pltpu.VMEM((NCHUNK, K), jnp.int32),
        pltpu.VMEM((NCHUNK, K), jnp.int32),
        pltpu.VMEM((K,), jnp.float32),
        pltpu.VMEM((640,), jnp.float32),
        pltpu.VMEM_SHARED((NS * 640,), jnp.float32),
        pltpu.VMEM_SHARED((NH_PAD + 8,), jnp.float32),
    ],
)
def _sc_degrees(src_hbm, dst_hbm, dego_hbm, degi_hbm,
                src_raw, dst_raw, src2d, ldst2d, ones, zbuf, dego_s, degi_s):
    c = lax.axis_index("c")
    s = lax.axis_index("s")
    base = c * NH

    one = jnp.ones((16,), jnp.float32)
    for k in range(K // 16):
        ones[pl.ds(16 * k, 16)] = one
    zv = jnp.zeros((16,), jnp.float32)
    for k in range(640 // 16):
        zbuf[pl.ds(16 * k, 16)] = zv
    pltpu.sync_copy(zbuf, dego_s.at[pl.ds(s * 640, 640)])
    pltpu.sync_copy(zbuf.at[pl.ds(0, 320)], degi_s.at[pl.ds(s * 320, 320)])

    @pl.when(s == 0)
    def _():
        pltpu.sync_copy(zbuf.at[pl.ds(0, 8)], degi_s.at[pl.ds(NH_PAD, 8)])

    pltpu.sync_copy(src_hbm.at[pl.ds(s * EPT, EPT)], src_raw)
    pltpu.sync_copy(dst_hbm.at[pl.ds(s * EPT, EPT)], dst_raw)

    @pl.loop(0, NCHUNK)
    def _(j):
        off = pl.multiple_of(j * K, K)
        for k in range(K // 16):
            sv = src_raw[pl.ds(off + 16 * k, 16)]
            src2d[j, pl.ds(16 * k, 16)] = sv
            dv = dst_raw[pl.ds(off + 16 * k, 16)]
            lv = dv - base
            ok = (lv >= 0) & (lv < NH)
            ldst2d[j, pl.ds(16 * k, 16)] = jnp.where(ok, lv, DUMMY)

    plsc.subcore_barrier()

    @pl.loop(0, NCHUNK)
    def _(j):
        pltpu.sync_copy(ones, degi_s.at[ldst2d.at[j]], add=True)

    @pl.when(c == 0)
    def _():
        @pl.loop(0, NCHUNK)
        def _(j):
            pltpu.sync_copy(ones, dego_s.at[src2d.at[j]], add=True)

    plsc.subcore_barrier()

    @pl.when(c == 0)
    def _():
        pltpu.sync_copy(dego_s.at[pl.ds(s * 640, 640)],
                        dego_hbm.at[pl.ds(s * 640, 640)])
    pltpu.sync_copy(degi_s.at[pl.ds(s * 320, 320)],
                    degi_hbm.at[c, pl.ds(s * 320, 320)])


@functools.partial(
    pl.kernel,
    out_type=jax.ShapeDtypeStruct((N, D), jnp.float32),
    mesh=_mesh,
    scratch_types=[
        pltpu.VMEM((EPT,), jnp.int32),
        pltpu.VMEM((EPT,), jnp.int32),
        pltpu.VMEM((NCHUNK, K), jnp.int32),
        pltpu.VMEM((K, D), jnp.float32),
        pltpu.VMEM_SHARED((AGG_ROWS, D), jnp.float32),
        pltpu.SemaphoreType.DMA,
    ],
)
def _sc_spmm(hw_hbm, src_hbm, dst_hbm, out_hbm,
             src_raw, dst_raw, ldst2d, rows, agg_s, sem):
    c = lax.axis_index("c")
    s = lax.axis_index("s")
    base = c * NH

    _zero_rows(rows)
    for q in range(4):
        pltpu.sync_copy(rows, agg_s.at[pl.ds(s * 320 + q * K, K)])

    pltpu.sync_copy(src_hbm.at[pl.ds(s * EPT, EPT)], src_raw)
    pltpu.sync_copy(dst_hbm.at[pl.ds(s * EPT, EPT)], dst_raw)

    @pl.loop(0, NCHUNK)
    def _(j):
        off = pl.multiple_of(j * K, K)
        for k in range(K // 16):
            dv = dst_raw[pl.ds(off + 16 * k, 16)]
            lv = dv - base
            ok = (lv >= 0) & (lv < NH)
            ldst2d[j, pl.ds(16 * k, 16)] = jnp.where(ok, lv, DUMMY)

    plsc.subcore_barrier()

    @pl.loop(0, NCHUNK)
    def _(j):
        off = pl.multiple_of(j * K, K)
        pltpu.async_copy(hw_hbm.at[src_raw.at[pl.ds(off, K)]], rows, sem).wait()
        pltpu.sync_copy(rows, agg_s.at[ldst2d.at[j]], add=True)

    plsc.subcore_barrier()

    @pl.when(s < NS - 1)
    def _():
        pltpu.sync_copy(agg_s.at[pl.ds(s * 320, 320)],
                        out_hbm.at[pl.ds(base + s * 320, 320)])

    @pl.when(s == NS - 1)
    def _():
        pltpu.sync_copy(agg_s.at[pl.ds((NS - 1) * 320, NH - (NS - 1) * 320)],
                        out_hbm.at[pl.ds(base + (NS - 1) * 320,
                                         NH - (NS - 1) * 320)])


_R = 400
_G = N // _R


def _tc_first_body(x_ref, w_ref, do_ref, o_ref):
    ns = lax.rsqrt(jnp.maximum(do_ref[...], 1.0))
    o_ref[...] = jnp.dot(x_ref[...], w_ref[...],
                         preferred_element_type=jnp.float32) * ns


def _tc_mid_body(a_ref, w_ref, b_ref, do_ref, di_ref, o_ref):
    ns = lax.rsqrt(jnp.maximum(do_ref[...], 1.0))
    nd = lax.rsqrt(jnp.maximum(di_ref[...], 1.0))
    h = jax.nn.relu(a_ref[...] * nd + b_ref[...])
    o_ref[...] = jnp.dot(h, w_ref[...],
                         preferred_element_type=jnp.float32) * ns


def _tc_head_body(a_ref, wt_ref, b_ref, bo_ref, di_ref, o_ref):
    nd = lax.rsqrt(jnp.maximum(di_ref[...], 1.0))
    h = jax.nn.relu(a_ref[...] * nd + b_ref[...])
    scores = jnp.sum(h * wt_ref[...], axis=1, keepdims=True) + bo_ref[0, 0]
    o_ref[...] = jax.nn.sigmoid(scores)


_row_spec = pl.BlockSpec((_R, D), lambda i: (i, 0))
_vec_spec = pl.BlockSpec((_R, 1), lambda i: (i, 0))
_w_spec = pl.BlockSpec((D, D), lambda i: (0, 0))
_b_spec = pl.BlockSpec((1, D), lambda i: (0, 0))
_out_f = jax.ShapeDtypeStruct((N, D), jnp.float32)
_out_v = jax.ShapeDtypeStruct((N, 1), jnp.float32)


def _tc_first(x, w, dego):
    return pl.pallas_call(
        _tc_first_body, grid=(_G,),
        in_specs=[_row_spec, _w_spec, _vec_spec],
        out_specs=_row_spec, out_shape=_out_f,
    )(x, w, dego)


def _tc_mid(agg, w, b, dego, degi):
    return pl.pallas_call(
        _tc_mid_body, grid=(_G,),
        in_specs=[_row_spec, _w_spec, _b_spec, _vec_spec, _vec_spec],
        out_specs=_row_spec, out_shape=_out_f,
    )(agg, w, b, dego, degi)


def _tc_head(agg, wt, b, bo, degi):
    return pl.pallas_call(
        _tc_head_body, grid=(_G,),
        in_specs=[_row_spec, _b_spec, _b_spec,
                  pl.BlockSpec((1, 1), lambda i: (0, 0)), _vec_spec],
        out_specs=_vec_spec, out_shape=_out_v,
    )(agg, wt, b, bo, degi)


def kernel(x, edge_index, W0, b0, W1, b1, W2, b2, Wout, bout):
    src = edge_index[0]
    dst = edge_index[1]

    dego_p, degi_p = _sc_degrees(src, dst)
    dego = dego_p[:N, None]
    degi = jnp.concatenate([degi_p[0, :NH], degi_p[1, :NH]])[:, None]

    b0r = b0[None, :]
    b1r = b1[None, :]
    b2r = b2[None, :]
    wt = Wout[:, 0][None, :]
    bo = bout[None, :]

    hw = _tc_first(x, W0, dego)
    agg = _sc_spmm(hw, src, dst)
    hw = _tc_mid(agg, W1, b0r, dego, degi)
    agg = _sc_spmm(hw, src, dst)
    hw = _tc_mid(agg, W2, b1r, dego, degi)
    agg = _sc_spmm(hw, src, dst)
    probs = _tc_head(agg, wt, b2r, bo, degi)
    return probs[:, 0]


# speedup vs baseline: 1.6746x; 1.6746x over previous
import functools

import jax
import jax.numpy as jnp
from jax import lax
from jax.experimental import pallas as pl
from jax.experimental.pallas import tpu as pltpu
from jax.experimental.pallas import tpu_sc as plsc

N = 10000
E = 160000
D = 256

NC = 2
NS = 16
NH = N // NC
DUMMY = NH
AGG_ROWS = NH + 8
EPT = E // NS
K = 80
NCHUNK = EPT // K

NW = NC * NS
EPW = E // NW
GFULL = EPW // K
GTAIL = EPW - GFULL * K

BK = 256
NBUCKET = 40
NCH = E // BK
NPAIR = 672

_mesh = plsc.VectorSubcoreMesh(core_axis_name="c", subcore_axis_name="s")


@functools.partial(
    pl.kernel,
    out_type=(
        jax.ShapeDtypeStruct((NS * 640,), jnp.float32),
        jax.ShapeDtypeStruct((NC * AGG_ROWS,), jnp.float32),
    ),
    mesh=_mesh,
    scratch_types=[
        pltpu.VMEM((EPT,), jnp.int32),
        pltpu.VMEM((EPT,), jnp.int32),
        pltpu.VMEM((NCHUNK, K), jnp.int32),
        pltpu.VMEM((NCHUNK, K), jnp.int32),
        pltpu.VMEM((K,), jnp.float32),
        pltpu.VMEM((640,), jnp.float32),
        pltpu.VMEM_SHARED((NS * 640,), jnp.float32),
        pltpu.VMEM_SHARED((AGG_ROWS,), jnp.float32),
    ],
)
def _sc_degrees(src_hbm, dst_hbm, dego_hbm, degi_hbm,
                src_raw, dst_raw, src2d, ldst2d, ones, zbuf, dego_s, degi_s):
    c = lax.axis_index("c")
    s = lax.axis_index("s")
    base = c * NH

    one = jnp.ones((16,), jnp.float32)
    for k in range(K // 16):
        ones[pl.ds(16 * k, 16)] = one
    zv = jnp.zeros((16,), jnp.float32)
    for k in range(640 // 16):
        zbuf[pl.ds(16 * k, 16)] = zv
    pltpu.sync_copy(zbuf, dego_s.at[pl.ds(s * 640, 640)])

    @pl.when(s < NS - 1)
    def _():
        pltpu.sync_copy(zbuf.at[pl.ds(0, 320)], degi_s.at[pl.ds(s * 320, 320)])

    @pl.when(s == NS - 1)
    def _():
        pltpu.sync_copy(zbuf.at[pl.ds(0, 208)],
                        degi_s.at[pl.ds((NS - 1) * 320, 208)])

    pltpu.sync_copy(src_hbm.at[pl.ds(s * EPT, EPT)], src_raw)
    pltpu.sync_copy(dst_hbm.at[pl.ds(s * EPT, EPT)], dst_raw)

    @pl.loop(0, NCHUNK)
    def _(j):
        off = pl.multiple_of(j * K, K)
        for k in range(K // 16):
            sv = src_raw[pl.ds(off + 16 * k, 16)]
            src2d[j, pl.ds(16 * k, 16)] = sv
            dv = dst_raw[pl.ds(off + 16 * k, 16)]
            lv = dv - base
            ok = (lv >= 0) & (lv < NH)
            ldst2d[j, pl.ds(16 * k, 16)] = jnp.where(ok, lv, DUMMY)

    plsc.subcore_barrier()

    @pl.loop(0, NCHUNK)
    def _(j):
        pltpu.sync_copy(ones, degi_s.at[ldst2d.at[j]], add=True)

    @pl.when(c == 0)
    def _():
        @pl.loop(0, NCHUNK)
        def _(j):
            pltpu.sync_copy(ones, dego_s.at[src2d.at[j]], add=True)

    plsc.subcore_barrier()

    @pl.when(c == 0)
    def _():
        pltpu.sync_copy(dego_s.at[pl.ds(s * 640, 640)], zbuf)
        pltpu.sync_copy(zbuf, dego_hbm.at[pl.ds(s * 640, 640)])

    @pl.when(s < NS - 1)
    def _():
        pltpu.sync_copy(degi_s.at[pl.ds(s * 320, 320)], zbuf.at[pl.ds(0, 320)])
        pltpu.sync_copy(zbuf.at[pl.ds(0, 320)],
                        degi_hbm.at[pl.ds(c * AGG_ROWS + s * 320, 320)])

    @pl.when(s == NS - 1)
    def _():
        pltpu.sync_copy(degi_s.at[pl.ds((NS - 1) * 320, 208)],
                        zbuf.at[pl.ds(0, 208)])
        pltpu.sync_copy(zbuf.at[pl.ds(0, 208)],
                        degi_hbm.at[pl.ds(c * AGG_ROWS + (NS - 1) * 320, 208)])


@functools.partial(
    pl.kernel,
    out_type=jax.ShapeDtypeStruct((E, D), jnp.float32),
    mesh=_mesh,
    scratch_types=[
        pltpu.VMEM((EPW,), jnp.int32),
        pltpu.VMEM((K, D), jnp.float32),
        pltpu.VMEM((K, D), jnp.float32),
        pltpu.SemaphoreType.DMA,
        pltpu.SemaphoreType.DMA,
        pltpu.SemaphoreType.DMA,
        pltpu.SemaphoreType.DMA,
    ],
)
def _sc_gather(hw_hbm, srcb_hbm, m_hbm, idx, rows0, rows1, g0, g1, w0, w1):
    c = lax.axis_index("c")
    s = lax.axis_index("s")
    wid = s * NC + c
    ebase = wid * EPW

    pltpu.sync_copy(srcb_hbm.at[pl.ds(ebase, EPW)], idx)

    rows = (rows0, rows1)
    gsem = (g0, g1)
    wsem = (w0, w1)

    def gather(j, b):
        off = pl.multiple_of(j * K, K)
        return pltpu.async_copy(hw_hbm.at[idx.at[pl.ds(off, K)]],
                                rows[b], gsem[b])

    def write(j, b):
        off = pl.multiple_of(j * K, K)
        return pltpu.async_copy(rows[b], m_hbm.at[pl.ds(ebase + off, K)],
                                wsem[b])

    def drain_write(b):
        pltpu.make_async_copy(rows[b], m_hbm.at[pl.ds(0, K)], wsem[b]).wait()

    def drain_gather(b):
        pltpu.make_async_copy(hw_hbm.at[idx.at[pl.ds(0, K)]],
                              rows[b], gsem[b]).wait()

    gather(0, 0).wait()
    gather(1, 1)
    write(0, 0)

    @pl.loop(0, (GFULL - 2) // 2)
    def _(jj):
        for t in range(2):
            j = 2 * jj + 2 + t
            b = t
            drain_write(b)
            drain_gather(1 - b)
            write(j - 1, 1 - b)
            gather(j, b)

    bl = GFULL % 2
    drain_gather(1 - bl)
    write(GFULL - 1, 1 - bl)

    drain_write(bl)
    toff = GFULL * K
    pltpu.async_copy(hw_hbm.at[idx.at[pl.ds(toff, GTAIL)]],
                     rows[bl].at[pl.ds(0, GTAIL)], gsem[bl]).wait()
    pltpu.sync_copy(rows[bl].at[pl.ds(0, GTAIL)],
                    m_hbm.at[pl.ds(ebase + toff, GTAIL)])
    drain_write(1 - bl)


def _tc_onehot_body(pc_ref, pb_ref, pf_ref, dstb_ref, m_ref, o_ref):
    i = pl.program_id(0)
    base = pb_ref[i] * BK
    dstv = dstb_ref[0, 0, :]
    rows_id = lax.broadcasted_iota(jnp.int32, (BK, BK), 0) + base
    oh = (rows_id == dstv[None, :]).astype(jnp.float32)

    @pl.when(pf_ref[i] == 1)
    def _():
        o_ref[...] = jnp.zeros((BK, D), jnp.float32)

    o_ref[...] += jnp.dot(oh, m_ref[...], preferred_element_type=jnp.float32)


def _tc_onehot(m, dstb3, pair_chunk, pair_bucket, pair_first):
    grid_spec = pltpu.PrefetchScalarGridSpec(
        num_scalar_prefetch=3,
        grid=(NPAIR,),
        in_specs=[
            pl.BlockSpec((1, 1, BK), lambda i, pc, pb, pf: (pc[i], 0, 0)),
            pl.BlockSpec((BK, D), lambda i, pc, pb, pf: (pc[i], 0)),
        ],
        out_specs=pl.BlockSpec((BK, D), lambda i, pc, pb, pf: (pb[i], 0)),
    )
    return pl.pallas_call(
        _tc_onehot_body,
        grid_spec=grid_spec,
        out_shape=jax.ShapeDtypeStruct(((NBUCKET + 1) * BK, D), jnp.float32),
        compiler_params=pltpu.CompilerParams(
            dimension_semantics=("arbitrary",)),
    )(pair_chunk, pair_bucket, pair_first, dstb3, m)


_R = 400
_G = N // _R


def _tc_first_body(x_ref, w_ref, do_ref, o_ref):
    ns = lax.rsqrt(jnp.maximum(do_ref[...], 1.0))
    o_ref[...] = jnp.dot(x_ref[...], w_ref[...],
                         preferred_element_type=jnp.float32) * ns


def _tc_mid_body(a_ref, w_ref, b_ref, do_ref, di_ref, o_ref):
    ns = lax.rsqrt(jnp.maximum(do_ref[...], 1.0))
    nd = lax.rsqrt(jnp.maximum(di_ref[...], 1.0))
    h = jax.nn.relu(a_ref[...] * nd + b_ref[...])
    o_ref[...] = jnp.dot(h, w_ref[...],
                         preferred_element_type=jnp.float32) * ns


def _tc_head_body(a_ref, wt_ref, b_ref, bo_ref, di_ref, o_ref):
    nd = lax.rsqrt(jnp.maximum(di_ref[...], 1.0))
    h = jax.nn.relu(a_ref[...] * nd + b_ref[...])
    scores = jnp.sum(h * wt_ref[...], axis=1, keepdims=True) + bo_ref[0, 0]
    o_ref[...] = jax.nn.sigmoid(scores)


_row_spec = pl.BlockSpec((_R, D), lambda i: (i, 0))
_vec_spec = pl.BlockSpec((_R, 1), lambda i: (i, 0))
_w_spec = pl.BlockSpec((D, D), lambda i: (0, 0))
_b_spec = pl.BlockSpec((1, D), lambda i: (0, 0))
_out_f = jax.ShapeDtypeStruct((N, D), jnp.float32)
_out_v = jax.ShapeDtypeStruct((N, 1), jnp.float32)


def _tc_first(x, w, dego):
    return pl.pallas_call(
        _tc_first_body, grid=(_G,),
        in_specs=[_row_spec, _w_spec, _vec_spec],
        out_specs=_row_spec, out_shape=_out_f,
    )(x, w, dego)


def _tc_mid(agg, w, b, dego, degi):
    return pl.pallas_call(
        _tc_mid_body, grid=(_G,),
        in_specs=[_row_spec, _w_spec, _b_spec, _vec_spec, _vec_spec],
        out_specs=_row_spec, out_shape=_out_f,
    )(agg, w, b, dego, degi)


def _tc_head(agg, wt, b, bo, degi):
    return pl.pallas_call(
        _tc_head_body, grid=(_G,),
        in_specs=[_row_spec, _b_spec, _b_spec,
                  pl.BlockSpec((1, 1), lambda i: (0, 0)), _vec_spec],
        out_specs=_vec_spec, out_shape=_out_v,
    )(agg, wt, b, bo, degi)


def kernel(x, edge_index, W0, b0, W1, b1, W2, b2, Wout, bout):
    src = edge_index[0]
    dst = edge_index[1]

    dego_p, degi_p = _sc_degrees(src, dst)
    dego = dego_p[:N, None]
    degi_vec = jnp.concatenate([degi_p[:NH], degi_p[AGG_ROWS:AGG_ROWS + NH]])
    degi = degi_vec[:, None]

    perm = jnp.argsort(dst)
    srcb = jnp.take(src, perm)
    dstb3 = jnp.take(dst, perm).reshape(NCH, 1, BK)

    dpad = jnp.concatenate([degi_vec, jnp.zeros((NBUCKET * BK - N,),
                                                jnp.float32)])
    bucket_sizes = dpad.reshape(NBUCKET, BK).sum(axis=1).astype(jnp.int32)
    off = jnp.concatenate([jnp.zeros((1,), jnp.int32),
                           jnp.cumsum(bucket_sizes)])
    nonempty = off[1:] > off[:-1]
    start_chunk = jnp.where(nonempty, off[:-1] // BK, 0)
    end_chunk = jnp.where(nonempty, (off[1:] - 1) // BK, 0)
    m_b = end_chunk - start_chunk + 1
    total = jnp.sum(m_b)
    mb_full = jnp.concatenate([m_b, (NPAIR - total)[None]])
    start_full = jnp.concatenate([start_chunk, jnp.zeros((1,), jnp.int32)])
    pair_bucket = jnp.repeat(jnp.arange(NBUCKET + 1, dtype=jnp.int32),
                             mb_full, total_repeat_length=NPAIR)
    p_off = jnp.concatenate([jnp.zeros((1,), jnp.int32),
                             jnp.cumsum(mb_full)])[:NBUCKET + 1]
    pos_in_bucket = (jnp.arange(NPAIR, dtype=jnp.int32)
                     - jnp.take(p_off, pair_bucket))
    pair_chunk = jnp.minimum(jnp.take(start_full, pair_bucket) + pos_in_bucket,
                             NCH - 1)
    pair_first = (pos_in_bucket == 0).astype(jnp.int32)

    b0r = b0[None, :]
    b1r = b1[None, :]
    b2r = b2[None, :]
    wt = Wout[:, 0][None, :]
    bo = bout[None, :]

    hw = _tc_first(x, W0, dego)
    m = _sc_gather(hw, srcb)
    agg = _tc_onehot(m, dstb3, pair_chunk, pair_bucket, pair_first)
    hw = _tc_mid(agg, W1, b0r, dego, degi)
    m = _sc_gather(hw, srcb)
    agg = _tc_onehot(m, dstb3, pair_chunk, pair_bucket, pair_first)
    hw = _tc_mid(agg, W2, b1r, dego, degi)
    m = _sc_gather(hw, srcb)
    agg = _tc_onehot(m, dstb3, pair_chunk, pair_bucket, pair_first)
    probs = _tc_head(agg, wt, b2r, bo, degi)
    return probs[:, 0]
